# baseline (speedup 1.0000x reference)
import functools

import jax
import jax.numpy as jnp
from jax import lax
from jax.experimental import pallas as pl
from jax.experimental.pallas import tpu as pltpu
from jax.experimental.pallas import tpu_sc as plsc

NC = 2
NS = 16
NW = NC * NS
CH = 128

EPS = 1e-5


def _sc_mesh():
    return plsc.VectorSubcoreMesh(core_axis_name="c", subcore_axis_name="s")


def _sc_gather(table, idx, D):
    E = idx.shape[0]
    assert E % CH == 0
    nch = E // CH
    iters = (nch + NW - 1) // NW
    dt = table.dtype

    @functools.partial(
        pl.kernel,
        out_type=jax.ShapeDtypeStruct((E, D), dt),
        mesh=_sc_mesh(),
        scratch_types=[
            pltpu.VMEM((CH,), jnp.int32),
            pltpu.VMEM((CH, D), dt),
            pltpu.SemaphoreType.DMA,
        ],
    )
    def k(table_hbm, idx_hbm, out_hbm, idx_v, rows_v, sem):
        wid = lax.axis_index("s") * NC + lax.axis_index("c")

        def body(j, carry):
            c = j * NW + wid

            @pl.when(c < nch)
            def _():
                off = c * CH
                pltpu.sync_copy(idx_hbm.at[pl.ds(off, CH)], idx_v)
                pltpu.async_copy(table_hbm.at[idx_v], rows_v, sem).wait()
                pltpu.sync_copy(rows_v, out_hbm.at[pl.ds(off, CH)])

            return carry

        lax.fori_loop(0, iters, body, 0)

    return k(table, idx)


def _sc_scatter_add(rows, dst, zeros_nw, n):
    E, W = rows.shape
    assert E % CH == 0
    nch = E // CH
    iters = (nch + NW - 1) // NW

    @functools.partial(
        pl.kernel,
        out_type=jax.ShapeDtypeStruct((NC, n, W), jnp.float32),
        mesh=_sc_mesh(),
        scratch_types=[
            pltpu.VMEM((CH,), jnp.int32),
            pltpu.VMEM((CH, W), jnp.float32),
            pltpu.VMEM_SHARED((n, W), jnp.float32),
        ],
    )
    def k(m_hbm, dst_hbm, zer_hbm, out_hbm, idx_v, rows_v, acc_sh):
        cid = lax.axis_index("c")
        sid = lax.axis_index("s")
        wid = sid * NC + cid

        @pl.when(sid == 0)
        def _():
            pltpu.sync_copy(zer_hbm, acc_sh)

        plsc.subcore_barrier()

        def body(j, carry):
            c = j * NW + wid

            @pl.when(c < nch)
            def _():
                off = c * CH
                pltpu.sync_copy(dst_hbm.at[pl.ds(off, CH)], idx_v)
                pltpu.sync_copy(m_hbm.at[pl.ds(off, CH)], rows_v)
                pltpu.sync_copy(rows_v, acc_sh.at[idx_v], add=True)

            return carry

        lax.fori_loop(0, iters, body, 0)
        plsc.subcore_barrier()

        @pl.when(sid == 0)
        def _():
            pltpu.sync_copy(acc_sh, out_hbm.at[cid])

    return k(rows, dst, zeros_nw)


def _edge1_body(xs_ref, ea_ref, w1_ref, b1_ref, w2q_ref, b2r_ref, bsel_ref,
                out_ref):
    xs = xs_ref[...].astype(jnp.bfloat16)
    h = jnp.maximum(
        jnp.dot(ea_ref[...], w1_ref[...], preferred_element_type=jnp.float32)
        + b1_ref[...], 0.0)
    y = jnp.dot(xs, w2q_ref[...], preferred_element_type=jnp.float32)
    ht = jnp.concatenate([h] * 16, axis=1)
    m = (jnp.dot((ht * y).astype(jnp.bfloat16), bsel_ref[...],
                 preferred_element_type=jnp.float32)
         + jnp.dot(xs, b2r_ref[...], preferred_element_type=jnp.float32))
    eb = m.shape[0]
    cnt_cols = jnp.where(
        lax.broadcasted_iota(jnp.int32, (eb, 16), 1) == 0, 1.0, 0.0)
    out_ref[...] = jnp.concatenate(
        [m, cnt_cols, jnp.zeros((eb, 96), jnp.float32)], axis=1)


def _edge2_body(hs_ref, ea_ref, w1_ref, b1_ref, w2q_ref, b2r_ref, bsel_ref,
                out_ref):
    hs = hs_ref[:, 0:16].astype(jnp.bfloat16)
    h = jnp.maximum(
        jnp.dot(ea_ref[...], w1_ref[...], preferred_element_type=jnp.float32)
        + b1_ref[...], 0.0)
    y = jnp.dot(hs, w2q_ref[...], preferred_element_type=jnp.float32)
    ht = jnp.concatenate([h] * 16, axis=1)
    m = (jnp.dot((ht * y).astype(jnp.bfloat16), bsel_ref[...],
                 preferred_element_type=jnp.float32)
         + jnp.dot(hs, b2r_ref[...], preferred_element_type=jnp.float32))
    out_ref[...] = jnp.concatenate(
        [m, jnp.zeros((m.shape[0], 112), jnp.float32)], axis=1)


def _node1_body(p_ref, x_ref, root_ref, bias_ref, bns_ref, bnb_ref,
                h_ref, inv_ref):
    p = p_ref[0] + p_ref[1]
    s = p[:, 0:16]
    cnt = p[:, 16:17]
    inv = 1.0 / jnp.maximum(cnt, 1.0)
    v = (s * inv
         + jnp.dot(x_ref[...], root_ref[...],
                   preferred_element_type=jnp.float32)
         + bias_ref[...])
    h = jnp.maximum(v * bns_ref[...] + bnb_ref[...], 0.0)
    nb = h.shape[0]
    h_ref[...] = jnp.concatenate([h, jnp.zeros((nb, 112), jnp.float32)],
                                 axis=1)
    inv_ref[...] = jnp.broadcast_to(inv, inv_ref.shape)


def _node2_body(p_ref, inv_ref, h1_ref, root_ref, bias_ref, bns_ref, bnb_ref,
                out_ref):
    s = p_ref[0, :, 0:16] + p_ref[1, :, 0:16]
    v = (s * inv_ref[...]
         + jnp.dot(h1_ref[:, 0:16], root_ref[...],
                   preferred_element_type=jnp.float32)
         + bias_ref[...])
    out_ref[...] = jnp.maximum(v * bns_ref[...] + bnb_ref[...], 0.0)


def _pool_body(h2_ref, b_ref, w1_ref, b1_ref, w2_ref, b2_ref, out_ref):
    n, _ = h2_ref.shape
    g = 64
    h2 = h2_ref[...]
    oh = jnp.where(
        b_ref[...] == lax.broadcasted_iota(jnp.int32, (n, g), 1), 1.0, 0.0)
    s = lax.dot_general(oh, h2, (((0,), (0,)), ((), ())),
                        preferred_element_type=jnp.float32)
    cnt16 = lax.dot_general(oh, jnp.ones((n, 16), jnp.float32),
                            (((0,), (0,)), ((), ())),
                            preferred_element_type=jnp.float32)
    xp = s / jnp.maximum(cnt16, 1.0)
    hm = jnp.maximum(
        jnp.dot(xp, w1_ref[...], preferred_element_type=jnp.float32)
        + b1_ref[...], 0.0)
    out_ref[...] = (
        jnp.dot(hm, w2_ref[...], preferred_element_type=jnp.float32)
        + b2_ref[...])


def _full(shape):
    return pl.BlockSpec(shape, lambda i: (0,) * len(shape))


def kernel(x, edge_index, edge_attr, batch,
           e1_w1, e1_b1, e1_w2, e1_b2, root1, bias1, bn1_g, bn1_b,
           e2_w1, e2_b1, e2_w2, e2_b2, root2, bias2, bn2_g, bn2_b,
           m_w1, m_b1, m_w2, m_b2):
    N, DF = x.shape
    E = edge_index.shape[1]
    H = 16
    HD = e1_w1.shape[1]
    G = 64
    src = edge_index[0]
    dst = edge_index[1]

    EB = 1600
    NB = 2000
    n_eblk = E // EB
    n_nblk = N // NB

    bf = jnp.bfloat16
    w2q1 = e1_w2.reshape(HD, DF, H).transpose(1, 2, 0).reshape(DF, H * HD)
    w2q1 = w2q1.astype(bf)
    b2r1 = e1_b2.reshape(DF, H).astype(bf)
    w2q2 = e2_w2.reshape(HD, H, H).transpose(1, 2, 0).reshape(H, H * HD)
    w2q2 = w2q2.astype(bf)
    b2r2 = e2_b2.reshape(H, H).astype(bf)
    bsel = jnp.repeat(jnp.eye(H, dtype=bf), HD, axis=0)
    bns1 = (bn1_g / jnp.sqrt(1.0 + EPS)).reshape(1, H)
    bns2 = (bn2_g / jnp.sqrt(1.0 + EPS)).reshape(1, H)
    row = lambda v: v.reshape(1, -1)
    zeros128 = jnp.zeros((N, DF), jnp.float32)

    xs = _sc_gather(x, src, DF)

    m1 = pl.pallas_call(
        _edge1_body,
        grid=(n_eblk,),
        in_specs=[
            pl.BlockSpec((EB, DF), lambda i: (i, 0)),
            pl.BlockSpec((EB, 16), lambda i: (i, 0)),
            _full((16, HD)), _full((1, HD)),
            _full((DF, H * HD)), _full((DF, H)), _full((H * HD, H)),
        ],
        out_specs=pl.BlockSpec((EB, DF), lambda i: (i, 0)),
        out_shape=jax.ShapeDtypeStruct((E, DF), jnp.float32),
    )(xs, edge_attr, e1_w1, row(e1_b1), w2q1, b2r1, bsel)

    p1 = _sc_scatter_add(m1, dst, zeros128, N)

    h1, inv16 = pl.pallas_call(
        _node1_body,
        grid=(n_nblk,),
        in_specs=[
            pl.BlockSpec((NC, NB, DF), lambda i: (0, i, 0)),
            pl.BlockSpec((NB, DF), lambda i: (i, 0)),
            _full((DF, H)), _full((1, H)), _full((1, H)), _full((1, H)),
        ],
        out_specs=[
            pl.BlockSpec((NB, DF), lambda i: (i, 0)),
            pl.BlockSpec((NB, H), lambda i: (i, 0)),
        ],
        out_shape=[
            jax.ShapeDtypeStruct((N, DF), jnp.float32),
            jax.ShapeDtypeStruct((N, H), jnp.float32),
        ],
    )(p1, x, root1, row(bias1), bns1, row(bn1_b))

    h1s = _sc_gather(h1, src, DF)

    m2 = pl.pallas_call(
        _edge2_body,
        grid=(n_eblk,),
        in_specs=[
            pl.BlockSpec((EB, DF), lambda i: (i, 0)),
            pl.BlockSpec((EB, 16), lambda i: (i, 0)),
            _full((16, HD)), _full((1, HD)),
            _full((H, H * HD)), _full((H, H)), _full((H * HD, H)),
        ],
        out_specs=pl.BlockSpec((EB, DF), lambda i: (i, 0)),
        out_shape=jax.ShapeDtypeStruct((E, DF), jnp.float32),
    )(h1s, edge_attr, e2_w1, row(e2_b1), w2q2, b2r2, bsel)

    p2 = _sc_scatter_add(m2, dst, zeros128, N)

    h2 = pl.pallas_call(
        _node2_body,
        grid=(n_nblk,),
        in_specs=[
            pl.BlockSpec((NC, NB, DF), lambda i: (0, i, 0)),
            pl.BlockSpec((NB, H), lambda i: (i, 0)),
            pl.BlockSpec((NB, DF), lambda i: (i, 0)),
            _full((H, H)), _full((1, H)), _full((1, H)), _full((1, H)),
        ],
        out_specs=pl.BlockSpec((NB, H), lambda i: (i, 0)),
        out_shape=jax.ShapeDtypeStruct((N, H), jnp.float32),
    )(p2, inv16, h1, root2, row(bias2), bns2, row(bn2_b))

    out = pl.pallas_call(
        _pool_body,
        in_specs=[
            pl.BlockSpec((N, H), lambda: (0, 0)),
            pl.BlockSpec((N, 1), lambda: (0, 0)),
            pl.BlockSpec((H, 8), lambda: (0, 0)),
            pl.BlockSpec((1, 8), lambda: (0, 0)),
            pl.BlockSpec((8, H), lambda: (0, 0)),
            pl.BlockSpec((1, H), lambda: (0, 0)),
        ],
        out_specs=pl.BlockSpec((G, H), lambda: (0, 0)),
        out_shape=jax.ShapeDtypeStruct((G, H), jnp.float32),
    )(h2, batch.reshape(N, 1), m_w1, row(m_b1), m_w2, row(m_b2))

    return out


# speedup vs baseline: 1.2034x; 1.2034x over previous
import functools

import jax
import jax.numpy as jnp
from jax import lax
from jax.experimental import pallas as pl
from jax.experimental.pallas import tpu as pltpu
from jax.experimental.pallas import tpu_sc as plsc

NC = 2
NS = 16
NW = NC * NS
CH = 128

EPS = 1e-5


def _sc_mesh():
    return plsc.VectorSubcoreMesh(core_axis_name="c", subcore_axis_name="s")


def _sc_gather(table, idx, D, base, ec):
    assert ec % CH == 0
    nch = ec // CH
    iters = (nch + NW - 1) // NW
    dt = table.dtype

    @functools.partial(
        pl.kernel,
        out_type=jax.ShapeDtypeStruct((ec, D), dt),
        mesh=_sc_mesh(),
        scratch_types=[
            pltpu.VMEM((CH,), jnp.int32),
            pltpu.VMEM((CH, D), dt),
            pltpu.SemaphoreType.DMA,
        ],
    )
    def k(table_hbm, idx_hbm, out_hbm, idx_v, rows_v, sem):
        wid = lax.axis_index("s") * NC + lax.axis_index("c")

        def body(j, carry):
            c = j * NW + wid

            @pl.when(c < nch)
            def _():
                off = c * CH
                pltpu.sync_copy(idx_hbm.at[pl.ds(base + off, CH)], idx_v)
                pltpu.async_copy(table_hbm.at[idx_v], rows_v, sem).wait()
                pltpu.sync_copy(rows_v, out_hbm.at[pl.ds(off, CH)])

            return carry

        lax.fori_loop(0, iters, body, 0)

    return k(table, idx)


def _sc_scatter_add(rows, dst, zeros_nw, n, base):
    ec, W = rows.shape
    assert ec % CH == 0
    nch = ec // CH
    iters = (nch + NW - 1) // NW

    @functools.partial(
        pl.kernel,
        out_type=jax.ShapeDtypeStruct((NC, n, W), jnp.float32),
        mesh=_sc_mesh(),
        scratch_types=[
            pltpu.VMEM((CH,), jnp.int32),
            pltpu.VMEM((CH, W), jnp.float32),
            pltpu.VMEM_SHARED((n, W), jnp.float32),
        ],
    )
    def k(m_hbm, dst_hbm, zer_hbm, out_hbm, idx_v, rows_v, acc_sh):
        cid = lax.axis_index("c")
        sid = lax.axis_index("s")
        wid = sid * NC + cid

        @pl.when(sid == 0)
        def _():
            pltpu.sync_copy(zer_hbm, acc_sh)

        plsc.subcore_barrier()

        def body(j, carry):
            c = j * NW + wid

            @pl.when(c < nch)
            def _():
                off = c * CH
                pltpu.sync_copy(dst_hbm.at[pl.ds(base + off, CH)], idx_v)
                pltpu.sync_copy(m_hbm.at[pl.ds(off, CH)], rows_v)
                pltpu.sync_copy(rows_v, acc_sh.at[idx_v], add=True)

            return carry

        lax.fori_loop(0, iters, body, 0)
        plsc.subcore_barrier()

        @pl.when(sid == 0)
        def _():
            pltpu.sync_copy(acc_sh, out_hbm.at[cid])

    return k(rows, dst, zeros_nw)


def _edge1_body(xs_ref, ea_ref, w1_ref, b1_ref, w2q_ref, b2r_ref, bsel_ref,
                out_ref):
    xs = xs_ref[...].astype(jnp.bfloat16)
    h = jnp.maximum(
        jnp.dot(ea_ref[...], w1_ref[...], preferred_element_type=jnp.float32)
        + b1_ref[...], 0.0)
    y = jnp.dot(xs, w2q_ref[...], preferred_element_type=jnp.float32)
    ht = jnp.concatenate([h] * 16, axis=1)
    m = (jnp.dot((ht * y).astype(jnp.bfloat16), bsel_ref[...],
                 preferred_element_type=jnp.float32)
         + jnp.dot(xs, b2r_ref[...], preferred_element_type=jnp.float32))
    eb = m.shape[0]
    cnt_cols = jnp.where(
        lax.broadcasted_iota(jnp.int32, (eb, 16), 1) == 0, 1.0, 0.0)
    out_ref[...] = jnp.concatenate(
        [m, cnt_cols, jnp.zeros((eb, 96), jnp.float32)], axis=1)


def _edge2_body(hs_ref, ea_ref, w1_ref, b1_ref, w2q_ref, b2r_ref, bsel_ref,
                out_ref):
    hs = hs_ref[:, 0:16].astype(jnp.bfloat16)
    h = jnp.maximum(
        jnp.dot(ea_ref[...], w1_ref[...], preferred_element_type=jnp.float32)
        + b1_ref[...], 0.0)
    y = jnp.dot(hs, w2q_ref[...], preferred_element_type=jnp.float32)
    ht = jnp.concatenate([h] * 16, axis=1)
    m = (jnp.dot((ht * y).astype(jnp.bfloat16), bsel_ref[...],
                 preferred_element_type=jnp.float32)
         + jnp.dot(hs, b2r_ref[...], preferred_element_type=jnp.float32))
    out_ref[...] = jnp.concatenate(
        [m, jnp.zeros((m.shape[0], 112), jnp.float32)], axis=1)


def _node1_body(pa_ref, pb_ref, x_ref, root_ref, bias_ref, bns_ref, bnb_ref,
                h_ref, inv_ref):
    p = pa_ref[0] + pa_ref[1] + pb_ref[0] + pb_ref[1]
    s = p[:, 0:16]
    cnt = p[:, 16:17]
    inv = 1.0 / jnp.maximum(cnt, 1.0)
    v = (s * inv
         + jnp.dot(x_ref[...], root_ref[...],
                   preferred_element_type=jnp.float32)
         + bias_ref[...])
    h = jnp.maximum(v * bns_ref[...] + bnb_ref[...], 0.0)
    nb = h.shape[0]
    h_ref[...] = jnp.concatenate([h, jnp.zeros((nb, 112), jnp.float32)],
                                 axis=1)
    inv_ref[...] = jnp.broadcast_to(inv, inv_ref.shape)


def _node2_body(pa_ref, pb_ref, inv_ref, h1_ref, root_ref, bias_ref, bns_ref,
                bnb_ref, out_ref):
    s = (pa_ref[0, :, 0:16] + pa_ref[1, :, 0:16]
         + pb_ref[0, :, 0:16] + pb_ref[1, :, 0:16])
    v = (s * inv_ref[...]
         + jnp.dot(h1_ref[:, 0:16], root_ref[...],
                   preferred_element_type=jnp.float32)
         + bias_ref[...])
    out_ref[...] = jnp.maximum(v * bns_ref[...] + bnb_ref[...], 0.0)


def _pool_body(h2_ref, b_ref, w1_ref, b1_ref, w2_ref, b2_ref, out_ref):
    n, _ = h2_ref.shape
    g = 64
    h2 = h2_ref[...]
    oh = jnp.where(
        b_ref[...] == lax.broadcasted_iota(jnp.int32, (n, g), 1), 1.0, 0.0)
    s = lax.dot_general(oh, h2, (((0,), (0,)), ((), ())),
                        preferred_element_type=jnp.float32)
    cnt16 = lax.dot_general(oh, jnp.ones((n, 16), jnp.float32),
                            (((0,), (0,)), ((), ())),
                            preferred_element_type=jnp.float32)
    xp = s / jnp.maximum(cnt16, 1.0)
    hm = jnp.maximum(
        jnp.dot(xp, w1_ref[...], preferred_element_type=jnp.float32)
        + b1_ref[...], 0.0)
    out_ref[...] = (
        jnp.dot(hm, w2_ref[...], preferred_element_type=jnp.float32)
        + b2_ref[...])


def _full(shape):
    return pl.BlockSpec(shape, lambda i: (0,) * len(shape))


def kernel(x, edge_index, edge_attr, batch,
           e1_w1, e1_b1, e1_w2, e1_b2, root1, bias1, bn1_g, bn1_b,
           e2_w1, e2_b1, e2_w2, e2_b2, root2, bias2, bn2_g, bn2_b,
           m_w1, m_b1, m_w2, m_b2):
    N, DF = x.shape
    E = edge_index.shape[1]
    H = 16
    HD = e1_w1.shape[1]
    G = 64
    src = edge_index[0]
    dst = edge_index[1]

    EB = 1600
    NB = 2000
    n_eblk = E // EB
    n_nblk = N // NB

    bf = jnp.bfloat16
    w2q1 = e1_w2.reshape(HD, DF, H).transpose(1, 2, 0).reshape(DF, H * HD)
    w2q1 = w2q1.astype(bf)
    b2r1 = e1_b2.reshape(DF, H).astype(bf)
    w2q2 = e2_w2.reshape(HD, H, H).transpose(1, 2, 0).reshape(H, H * HD)
    w2q2 = w2q2.astype(bf)
    b2r2 = e2_b2.reshape(H, H).astype(bf)
    bsel = jnp.repeat(jnp.eye(H, dtype=bf), HD, axis=0)
    bns1 = (bn1_g / jnp.sqrt(1.0 + EPS)).reshape(1, H)
    bns2 = (bn2_g / jnp.sqrt(1.0 + EPS)).reshape(1, H)
    row = lambda v: v.reshape(1, -1)
    zeros128 = jnp.zeros((N, DF), jnp.float32)

    EC = E // 2
    n_eblk_c = EC // EB

    def edge_call(body, xs_c, w1, b1, w2q, b2r, din):
        return pl.pallas_call(
            body,
            grid=(n_eblk_c,),
            in_specs=[
                pl.BlockSpec((EB, DF), lambda i: (i, 0)),
                pl.BlockSpec((EB, 16), lambda i: (i, 0)),
                _full((16, HD)), _full((1, HD)),
                _full((din, H * HD)), _full((din, H)), _full((H * HD, H)),
            ],
            out_specs=pl.BlockSpec((EB, DF), lambda i: (i, 0)),
            out_shape=jax.ShapeDtypeStruct((EC, DF), jnp.float32),
        )(xs_c, edge_attr_c, w1, b1, w2q, b2r, bsel)

    p1 = []
    for c in range(2):
        base = c * EC
        edge_attr_c = lax.slice_in_dim(edge_attr, base, base + EC, axis=0)
        xs_c = _sc_gather(x, src, DF, base, EC)
        m1_c = edge_call(_edge1_body, xs_c, e1_w1, row(e1_b1), w2q1, b2r1, DF)
        p1.append(_sc_scatter_add(m1_c, dst, zeros128, N, base))

    h1, inv16 = pl.pallas_call(
        _node1_body,
        grid=(n_nblk,),
        in_specs=[
            pl.BlockSpec((NC, NB, DF), lambda i: (0, i, 0)),
            pl.BlockSpec((NC, NB, DF), lambda i: (0, i, 0)),
            pl.BlockSpec((NB, DF), lambda i: (i, 0)),
            _full((DF, H)), _full((1, H)), _full((1, H)), _full((1, H)),
        ],
        out_specs=[
            pl.BlockSpec((NB, DF), lambda i: (i, 0)),
            pl.BlockSpec((NB, H), lambda i: (i, 0)),
        ],
        out_shape=[
            jax.ShapeDtypeStruct((N, DF), jnp.float32),
            jax.ShapeDtypeStruct((N, H), jnp.float32),
        ],
    )(p1[0], p1[1], x, root1, row(bias1), bns1, row(bn1_b))

    p2 = []
    for c in range(2):
        base = c * EC
        edge_attr_c = lax.slice_in_dim(edge_attr, base, base + EC, axis=0)
        h1s_c = _sc_gather(h1, src, DF, base, EC)
        m2_c = edge_call(_edge2_body, h1s_c, e2_w1, row(e2_b1), w2q2, b2r2, H)
        p2.append(_sc_scatter_add(m2_c, dst, zeros128, N, base))

    h2 = pl.pallas_call(
        _node2_body,
        grid=(n_nblk,),
        in_specs=[
            pl.BlockSpec((NC, NB, DF), lambda i: (0, i, 0)),
            pl.BlockSpec((NC, NB, DF), lambda i: (0, i, 0)),
            pl.BlockSpec((NB, H), lambda i: (i, 0)),
            pl.BlockSpec((NB, DF), lambda i: (i, 0)),
            _full((H, H)), _full((1, H)), _full((1, H)), _full((1, H)),
        ],
        out_specs=pl.BlockSpec((NB, H), lambda i: (i, 0)),
        out_shape=jax.ShapeDtypeStruct((N, H), jnp.float32),
    )(p2[0], p2[1], inv16, h1, root2, row(bias2), bns2, row(bn2_b))

    out = pl.pallas_call(
        _pool_body,
        in_specs=[
            pl.BlockSpec((N, H), lambda: (0, 0)),
            pl.BlockSpec((N, 1), lambda: (0, 0)),
            pl.BlockSpec((H, 8), lambda: (0, 0)),
            pl.BlockSpec((1, 8), lambda: (0, 0)),
            pl.BlockSpec((8, H), lambda: (0, 0)),
            pl.BlockSpec((1, H), lambda: (0, 0)),
        ],
        out_specs=pl.BlockSpec((G, H), lambda: (0, 0)),
        out_shape=jax.ShapeDtypeStruct((G, H), jnp.float32),
    )(h2, batch.reshape(N, 1), m_w1, row(m_b1), m_w2, row(m_b2))

    return out


# speedup vs baseline: 1.2356x; 1.0268x over previous
import functools

import jax
import jax.numpy as jnp
from jax import lax
from jax.experimental import pallas as pl
from jax.experimental.pallas import tpu as pltpu
from jax.experimental.pallas import tpu_sc as plsc

NC = 2
NS = 16
NW = NC * NS
CH = 128

EPS = 1e-5


def _sc_mesh():
    return plsc.VectorSubcoreMesh(core_axis_name="c", subcore_axis_name="s")


def _sc_gather(table, idx, D, base, ec):
    assert ec % CH == 0
    nch = ec // CH
    iters = (nch + NW - 1) // NW
    dt = table.dtype

    @functools.partial(
        pl.kernel,
        out_type=jax.ShapeDtypeStruct((ec, D), dt),
        mesh=_sc_mesh(),
        scratch_types=[
            pltpu.VMEM((CH,), jnp.int32),
            pltpu.VMEM((CH, D), dt),
            pltpu.SemaphoreType.DMA,
        ],
    )
    def k(table_hbm, idx_hbm, out_hbm, idx_v, rows_v, sem):
        wid = lax.axis_index("s") * NC + lax.axis_index("c")

        def body(j, carry):
            c = j * NW + wid

            @pl.when(c < nch)
            def _():
                off = c * CH
                pltpu.sync_copy(idx_hbm.at[pl.ds(base + off, CH)], idx_v)
                pltpu.async_copy(table_hbm.at[idx_v], rows_v, sem).wait()
                pltpu.sync_copy(rows_v, out_hbm.at[pl.ds(off, CH)])

            return carry

        lax.fori_loop(0, iters, body, 0)

    return k(table, idx)


def _sc_scatter_add(rows, dst, zeros_nw, n, base):
    ec, W = rows.shape
    assert ec % CH == 0
    nch = ec // CH
    iters = (nch + NW - 1) // NW

    @functools.partial(
        pl.kernel,
        out_type=jax.ShapeDtypeStruct((NC, n, W), jnp.float32),
        mesh=_sc_mesh(),
        scratch_types=[
            pltpu.VMEM((CH,), jnp.int32),
            pltpu.VMEM((CH, W), jnp.float32),
            pltpu.VMEM_SHARED((n, W), jnp.float32),
        ],
    )
    def k(m_hbm, dst_hbm, zer_hbm, out_hbm, idx_v, rows_v, acc_sh):
        cid = lax.axis_index("c")
        sid = lax.axis_index("s")
        wid = sid * NC + cid

        @pl.when(sid == 0)
        def _():
            pltpu.sync_copy(zer_hbm, acc_sh)

        plsc.subcore_barrier()

        def body(j, carry):
            c = j * NW + wid

            @pl.when(c < nch)
            def _():
                off = c * CH
                pltpu.sync_copy(dst_hbm.at[pl.ds(base + off, CH)], idx_v)
                pltpu.sync_copy(m_hbm.at[pl.ds(off, CH)], rows_v)
                pltpu.sync_copy(rows_v, acc_sh.at[idx_v], add=True)

            return carry

        lax.fori_loop(0, iters, body, 0)
        plsc.subcore_barrier()

        @pl.when(sid == 0)
        def _():
            pltpu.sync_copy(acc_sh, out_hbm.at[cid])

    return k(rows, dst, zeros_nw)


def _edge1_body(xs_ref, ea_ref, w1_ref, b1_ref, w2q_ref, b2r_ref, bsel_ref,
                out_ref):
    xs = xs_ref[...].astype(jnp.bfloat16)
    h = jnp.maximum(
        jnp.dot(ea_ref[...], w1_ref[...], preferred_element_type=jnp.float32)
        + b1_ref[...], 0.0)
    y = jnp.dot(xs, w2q_ref[...], preferred_element_type=jnp.float32)
    ht = jnp.concatenate([h] * 16, axis=1)
    m = (jnp.dot((ht * y).astype(jnp.bfloat16), bsel_ref[...],
                 preferred_element_type=jnp.float32)
         + jnp.dot(xs, b2r_ref[...], preferred_element_type=jnp.float32))
    eb = m.shape[0]
    cnt_cols = jnp.where(
        lax.broadcasted_iota(jnp.int32, (eb, 16), 1) == 0, 1.0, 0.0)
    out_ref[...] = jnp.concatenate(
        [m, cnt_cols, jnp.zeros((eb, 96), jnp.float32)], axis=1)


def _edge2_body(hs_ref, ea_ref, w1_ref, b1_ref, w2q_ref, b2r_ref, bsel_ref,
                out_ref):
    hs = hs_ref[:, 0:16].astype(jnp.bfloat16)
    h = jnp.maximum(
        jnp.dot(ea_ref[...], w1_ref[...], preferred_element_type=jnp.float32)
        + b1_ref[...], 0.0)
    y = jnp.dot(hs, w2q_ref[...], preferred_element_type=jnp.float32)
    ht = jnp.concatenate([h] * 16, axis=1)
    m = (jnp.dot((ht * y).astype(jnp.bfloat16), bsel_ref[...],
                 preferred_element_type=jnp.float32)
         + jnp.dot(hs, b2r_ref[...], preferred_element_type=jnp.float32))
    out_ref[...] = jnp.concatenate(
        [m, jnp.zeros((m.shape[0], 112), jnp.float32)], axis=1)


def _node1_body(pa_ref, pb_ref, x_ref, root_ref, bias_ref, bns_ref, bnb_ref,
                h_ref, inv_ref):
    p = pa_ref[0] + pa_ref[1] + pb_ref[0] + pb_ref[1]
    s = p[:, 0:16]
    cnt = p[:, 16:17]
    inv = 1.0 / jnp.maximum(cnt, 1.0)
    v = (s * inv
         + jnp.dot(x_ref[...], root_ref[...],
                   preferred_element_type=jnp.float32)
         + bias_ref[...])
    h = jnp.maximum(v * bns_ref[...] + bnb_ref[...], 0.0)
    nb = h.shape[0]
    h_ref[...] = jnp.concatenate([h, jnp.zeros((nb, 112), jnp.float32)],
                                 axis=1)
    inv_ref[...] = jnp.broadcast_to(inv, inv_ref.shape)


def _node2_body(pa_ref, pb_ref, inv_ref, h1_ref, root_ref, bias_ref, bns_ref,
                bnb_ref, out_ref):
    s = (pa_ref[0, :, 0:16] + pa_ref[1, :, 0:16]
         + pb_ref[0, :, 0:16] + pb_ref[1, :, 0:16])
    v = (s * inv_ref[...]
         + jnp.dot(h1_ref[:, 0:16], root_ref[...],
                   preferred_element_type=jnp.float32)
         + bias_ref[...])
    out_ref[...] = jnp.maximum(v * bns_ref[...] + bnb_ref[...], 0.0)


def _pool_body(h2_ref, b_ref, w1_ref, b1_ref, w2_ref, b2_ref, out_ref):
    n, _ = h2_ref.shape
    g = 64
    h2 = h2_ref[...]
    oh = jnp.where(
        b_ref[...] == lax.broadcasted_iota(jnp.int32, (n, g), 1), 1.0, 0.0)
    s = lax.dot_general(oh, h2, (((0,), (0,)), ((), ())),
                        preferred_element_type=jnp.float32)
    cnt16 = lax.dot_general(oh, jnp.ones((n, 16), jnp.float32),
                            (((0,), (0,)), ((), ())),
                            preferred_element_type=jnp.float32)
    xp = s / jnp.maximum(cnt16, 1.0)
    hm = jnp.maximum(
        jnp.dot(xp, w1_ref[...], preferred_element_type=jnp.float32)
        + b1_ref[...], 0.0)
    out_ref[...] = (
        jnp.dot(hm, w2_ref[...], preferred_element_type=jnp.float32)
        + b2_ref[...])


def _full(shape):
    return pl.BlockSpec(shape, lambda i: (0,) * len(shape))


def kernel(x, edge_index, edge_attr, batch,
           e1_w1, e1_b1, e1_w2, e1_b2, root1, bias1, bn1_g, bn1_b,
           e2_w1, e2_b1, e2_w2, e2_b2, root2, bias2, bn2_g, bn2_b,
           m_w1, m_b1, m_w2, m_b2):
    N, DF = x.shape
    E = edge_index.shape[1]
    H = 16
    HD = e1_w1.shape[1]
    G = 64
    src = edge_index[0]
    dst = edge_index[1]

    EB = 1600
    NB = 2000
    n_eblk = E // EB
    n_nblk = N // NB

    bf = jnp.bfloat16
    w2q1 = e1_w2.reshape(HD, DF, H).transpose(1, 2, 0).reshape(DF, H * HD)
    w2q1 = w2q1.astype(bf)
    b2r1 = e1_b2.reshape(DF, H).astype(bf)
    w2q2 = e2_w2.reshape(HD, H, H).transpose(1, 2, 0).reshape(H, H * HD)
    w2q2 = w2q2.astype(bf)
    b2r2 = e2_b2.reshape(H, H).astype(bf)
    bsel = jnp.repeat(jnp.eye(H, dtype=bf), HD, axis=0)
    bns1 = (bn1_g / jnp.sqrt(1.0 + EPS)).reshape(1, H)
    bns2 = (bn2_g / jnp.sqrt(1.0 + EPS)).reshape(1, H)
    row = lambda v: v.reshape(1, -1)
    zeros128 = jnp.zeros((N, DF), jnp.float32)

    EC = E // 2
    n_eblk_c = EC // EB

    def edge_call(body, xs_c, w1, b1, w2q, b2r, din, c):
        off_blk = c * n_eblk_c
        return pl.pallas_call(
            body,
            grid=(n_eblk_c,),
            in_specs=[
                pl.BlockSpec((EB, DF), lambda i: (i, 0)),
                pl.BlockSpec((EB, 16), lambda i: (i + off_blk, 0)),
                _full((16, HD)), _full((1, HD)),
                _full((din, H * HD)), _full((din, H)), _full((H * HD, H)),
            ],
            out_specs=pl.BlockSpec((EB, DF), lambda i: (i, 0)),
            out_shape=jax.ShapeDtypeStruct((EC, DF), jnp.float32),
        )(xs_c, edge_attr, w1, b1, w2q, b2r, bsel)

    p1 = []
    for c in range(2):
        base = c * EC
        xs_c = _sc_gather(x, src, DF, base, EC)
        m1_c = edge_call(_edge1_body, xs_c, e1_w1, row(e1_b1), w2q1, b2r1,
                         DF, c)
        p1.append(_sc_scatter_add(m1_c, dst, zeros128, N, base))

    h1, inv16 = pl.pallas_call(
        _node1_body,
        grid=(n_nblk,),
        in_specs=[
            pl.BlockSpec((NC, NB, DF), lambda i: (0, i, 0)),
            pl.BlockSpec((NC, NB, DF), lambda i: (0, i, 0)),
            pl.BlockSpec((NB, DF), lambda i: (i, 0)),
            _full((DF, H)), _full((1, H)), _full((1, H)), _full((1, H)),
        ],
        out_specs=[
            pl.BlockSpec((NB, DF), lambda i: (i, 0)),
            pl.BlockSpec((NB, H), lambda i: (i, 0)),
        ],
        out_shape=[
            jax.ShapeDtypeStruct((N, DF), jnp.float32),
            jax.ShapeDtypeStruct((N, H), jnp.float32),
        ],
    )(p1[0], p1[1], x, root1, row(bias1), bns1, row(bn1_b))

    p2 = []
    for c in range(2):
        base = c * EC
        h1s_c = _sc_gather(h1, src, DF, base, EC)
        m2_c = edge_call(_edge2_body, h1s_c, e2_w1, row(e2_b1), w2q2, b2r2,
                         H, c)
        p2.append(_sc_scatter_add(m2_c, dst, zeros128, N, base))

    h2 = pl.pallas_call(
        _node2_body,
        grid=(n_nblk,),
        in_specs=[
            pl.BlockSpec((NC, NB, DF), lambda i: (0, i, 0)),
            pl.BlockSpec((NC, NB, DF), lambda i: (0, i, 0)),
            pl.BlockSpec((NB, H), lambda i: (i, 0)),
            pl.BlockSpec((NB, DF), lambda i: (i, 0)),
            _full((H, H)), _full((1, H)), _full((1, H)), _full((1, H)),
        ],
        out_specs=pl.BlockSpec((NB, H), lambda i: (i, 0)),
        out_shape=jax.ShapeDtypeStruct((N, H), jnp.float32),
    )(p2[0], p2[1], inv16, h1, root2, row(bias2), bns2, row(bn2_b))

    out = pl.pallas_call(
        _pool_body,
        in_specs=[
            pl.BlockSpec((N, H), lambda: (0, 0)),
            pl.BlockSpec((N, 1), lambda: (0, 0)),
            pl.BlockSpec((H, 8), lambda: (0, 0)),
            pl.BlockSpec((1, 8), lambda: (0, 0)),
            pl.BlockSpec((8, H), lambda: (0, 0)),
            pl.BlockSpec((1, H), lambda: (0, 0)),
        ],
        out_specs=pl.BlockSpec((G, H), lambda: (0, 0)),
        out_shape=jax.ShapeDtypeStruct((G, H), jnp.float32),
    )(h2, batch.reshape(N, 1), m_w1, row(m_b1), m_w2, row(m_b2))

    return out


# speedup vs baseline: 1.2863x; 1.0410x over previous
import functools

import jax
import jax.numpy as jnp
from jax import lax
from jax.experimental import pallas as pl
from jax.experimental.pallas import tpu as pltpu
from jax.experimental.pallas import tpu_sc as plsc

NC = 2
NS = 16
NW = NC * NS
CH = 128

EPS = 1e-5


def _sc_mesh():
    return plsc.VectorSubcoreMesh(core_axis_name="c", subcore_axis_name="s")


def _sc_gather(table, idx, D, base, ec):
    assert ec % CH == 0
    nch = ec // CH
    iters = nch // NW
    dt = table.dtype

    @functools.partial(
        pl.kernel,
        out_type=jax.ShapeDtypeStruct((ec, D), dt),
        mesh=_sc_mesh(),
        scratch_types=[
            pltpu.VMEM((CH,), jnp.int32),
            pltpu.VMEM((CH,), jnp.int32),
            pltpu.VMEM((CH, D), dt),
            pltpu.VMEM((CH, D), dt),
            pltpu.SemaphoreType.DMA,
            pltpu.SemaphoreType.DMA,
            pltpu.SemaphoreType.DMA,
            pltpu.SemaphoreType.DMA,
        ],
    )
    def k(table_hbm, idx_hbm, out_hbm, i0, i1, r0, r1, si, sg, sw0, sw1):
        wid = lax.axis_index("s") * NC + lax.axis_index("c")
        idx_v = [i0, i1]
        rows_v = [r0, r1]
        sw = [sw0, sw1]
        wpend = [None, None]

        def chunk_off(j):
            return (j * NW + wid) * CH

        pltpu.sync_copy(idx_hbm.at[pl.ds(base + chunk_off(0), CH)], idx_v[0])
        for j in range(iters):
            b = j % 2
            hi = None
            if j + 1 < iters:
                hi = pltpu.async_copy(
                    idx_hbm.at[pl.ds(base + chunk_off(j + 1), CH)],
                    idx_v[1 - b], si)
            if wpend[b] is not None:
                wpend[b].wait()
            pltpu.async_copy(table_hbm.at[idx_v[b]], rows_v[b], sg).wait()
            wpend[b] = pltpu.async_copy(
                rows_v[b], out_hbm.at[pl.ds(chunk_off(j), CH)], sw[b])
            if hi is not None:
                hi.wait()
        for b in range(2):
            if wpend[b] is not None:
                wpend[b].wait()

        if iters * NW < nch:
            @pl.when(iters * NW + wid < nch)
            def _():
                off = chunk_off(iters)
                pltpu.sync_copy(idx_hbm.at[pl.ds(base + off, CH)], idx_v[0])
                pltpu.async_copy(table_hbm.at[idx_v[0]], rows_v[0], sg).wait()
                pltpu.sync_copy(rows_v[0], out_hbm.at[pl.ds(off, CH)])

    return k(table, idx)


def _sc_scatter_add(rows, dst, zeros_nw, n, base):
    ec, W = rows.shape
    assert ec % CH == 0
    nch = ec // CH
    iters = nch // NW

    @functools.partial(
        pl.kernel,
        out_type=jax.ShapeDtypeStruct((NC, n, W), jnp.float32),
        mesh=_sc_mesh(),
        scratch_types=[
            pltpu.VMEM((CH,), jnp.int32),
            pltpu.VMEM((CH,), jnp.int32),
            pltpu.VMEM((CH, W), jnp.float32),
            pltpu.VMEM((CH, W), jnp.float32),
            pltpu.VMEM_SHARED((n, W), jnp.float32),
            pltpu.SemaphoreType.DMA,
            pltpu.SemaphoreType.DMA,
            pltpu.SemaphoreType.DMA,
            pltpu.SemaphoreType.DMA,
        ],
    )
    def k(m_hbm, dst_hbm, zer_hbm, out_hbm, i0, i1, r0, r1, acc_sh,
          si, sm, ss0, ss1):
        cid = lax.axis_index("c")
        sid = lax.axis_index("s")
        wid = sid * NC + cid
        idx_v = [i0, i1]
        rows_v = [r0, r1]
        ss = [ss0, ss1]
        spend = [None, None]

        def chunk_off(j):
            return (j * NW + wid) * CH

        @pl.when(sid == 0)
        def _():
            pltpu.sync_copy(zer_hbm, acc_sh)

        plsc.subcore_barrier()

        pltpu.sync_copy(dst_hbm.at[pl.ds(base + chunk_off(0), CH)], idx_v[0])
        pltpu.sync_copy(m_hbm.at[pl.ds(chunk_off(0), CH)], rows_v[0])
        for j in range(iters):
            b = j % 2
            hi = hr = None
            if j + 1 < iters:
                if spend[1 - b] is not None:
                    spend[1 - b].wait()
                    spend[1 - b] = None
                hi = pltpu.async_copy(
                    dst_hbm.at[pl.ds(base + chunk_off(j + 1), CH)],
                    idx_v[1 - b], si)
                hr = pltpu.async_copy(
                    m_hbm.at[pl.ds(chunk_off(j + 1), CH)], rows_v[1 - b], sm)
            spend[b] = pltpu.async_copy(
                rows_v[b], acc_sh.at[idx_v[b]], ss[b], add=True)
            if hi is not None:
                hi.wait()
                hr.wait()
        for b in range(2):
            if spend[b] is not None:
                spend[b].wait()

        if iters * NW < nch:
            @pl.when(iters * NW + wid < nch)
            def _():
                off = chunk_off(iters)
                pltpu.sync_copy(dst_hbm.at[pl.ds(base + off, CH)], idx_v[0])
                pltpu.sync_copy(m_hbm.at[pl.ds(off, CH)], rows_v[0])
                pltpu.sync_copy(rows_v[0], acc_sh.at[idx_v[0]], add=True)

        plsc.subcore_barrier()

        @pl.when(sid == 0)
        def _():
            pltpu.sync_copy(acc_sh, out_hbm.at[cid])

    return k(rows, dst, zeros_nw)


def _edge1_body(xs_ref, ea_ref, w1_ref, b1_ref, w2q_ref, b2r_ref, bsel_ref,
                out_ref):
    xs = xs_ref[...].astype(jnp.bfloat16)
    h = jnp.maximum(
        jnp.dot(ea_ref[...], w1_ref[...], preferred_element_type=jnp.float32)
        + b1_ref[...], 0.0)
    y = jnp.dot(xs, w2q_ref[...], preferred_element_type=jnp.float32)
    ht = jnp.concatenate([h] * 16, axis=1)
    m = (jnp.dot((ht * y).astype(jnp.bfloat16), bsel_ref[...],
                 preferred_element_type=jnp.float32)
         + jnp.dot(xs, b2r_ref[...], preferred_element_type=jnp.float32))
    eb = m.shape[0]
    cnt_cols = jnp.where(
        lax.broadcasted_iota(jnp.int32, (eb, 16), 1) == 0, 1.0, 0.0)
    out_ref[...] = jnp.concatenate(
        [m, cnt_cols, jnp.zeros((eb, 96), jnp.float32)], axis=1)


def _edge2_body(hs_ref, ea_ref, w1_ref, b1_ref, w2q_ref, b2r_ref, bsel_ref,
                out_ref):
    hs = hs_ref[:, 0:16].astype(jnp.bfloat16)
    h = jnp.maximum(
        jnp.dot(ea_ref[...], w1_ref[...], preferred_element_type=jnp.float32)
        + b1_ref[...], 0.0)
    y = jnp.dot(hs, w2q_ref[...], preferred_element_type=jnp.float32)
    ht = jnp.concatenate([h] * 16, axis=1)
    m = (jnp.dot((ht * y).astype(jnp.bfloat16), bsel_ref[...],
                 preferred_element_type=jnp.float32)
         + jnp.dot(hs, b2r_ref[...], preferred_element_type=jnp.float32))
    out_ref[...] = jnp.concatenate(
        [m, jnp.zeros((m.shape[0], 112), jnp.float32)], axis=1)


def _node1_body(pa_ref, pb_ref, x_ref, root_ref, bias_ref, bns_ref, bnb_ref,
                h_ref, inv_ref):
    p = pa_ref[0] + pa_ref[1] + pb_ref[0] + pb_ref[1]
    s = p[:, 0:16]
    cnt = p[:, 16:17]
    inv = 1.0 / jnp.maximum(cnt, 1.0)
    v = (s * inv
         + jnp.dot(x_ref[...], root_ref[...],
                   preferred_element_type=jnp.float32)
         + bias_ref[...])
    h = jnp.maximum(v * bns_ref[...] + bnb_ref[...], 0.0)
    nb = h.shape[0]
    h_ref[...] = jnp.concatenate([h, jnp.zeros((nb, 112), jnp.float32)],
                                 axis=1)
    inv_ref[...] = jnp.broadcast_to(inv, inv_ref.shape)


def _node2_body(pa_ref, pb_ref, inv_ref, h1_ref, root_ref, bias_ref, bns_ref,
                bnb_ref, out_ref):
    s = (pa_ref[0, :, 0:16] + pa_ref[1, :, 0:16]
         + pb_ref[0, :, 0:16] + pb_ref[1, :, 0:16])
    v = (s * inv_ref[...]
         + jnp.dot(h1_ref[:, 0:16], root_ref[...],
                   preferred_element_type=jnp.float32)
         + bias_ref[...])
    out_ref[...] = jnp.maximum(v * bns_ref[...] + bnb_ref[...], 0.0)


def _pool_body(h2_ref, b_ref, w1_ref, b1_ref, w2_ref, b2_ref, out_ref):
    n, _ = h2_ref.shape
    g = 64
    h2 = h2_ref[...]
    oh = jnp.where(
        b_ref[...] == lax.broadcasted_iota(jnp.int32, (n, g), 1), 1.0, 0.0)
    s = lax.dot_general(oh, h2, (((0,), (0,)), ((), ())),
                        preferred_element_type=jnp.float32)
    cnt16 = lax.dot_general(oh, jnp.ones((n, 16), jnp.float32),
                            (((0,), (0,)), ((), ())),
                            preferred_element_type=jnp.float32)
    xp = s / jnp.maximum(cnt16, 1.0)
    hm = jnp.maximum(
        jnp.dot(xp, w1_ref[...], preferred_element_type=jnp.float32)
        + b1_ref[...], 0.0)
    out_ref[...] = (
        jnp.dot(hm, w2_ref[...], preferred_element_type=jnp.float32)
        + b2_ref[...])


def _full(shape):
    return pl.BlockSpec(shape, lambda i: (0,) * len(shape))


def kernel(x, edge_index, edge_attr, batch,
           e1_w1, e1_b1, e1_w2, e1_b2, root1, bias1, bn1_g, bn1_b,
           e2_w1, e2_b1, e2_w2, e2_b2, root2, bias2, bn2_g, bn2_b,
           m_w1, m_b1, m_w2, m_b2):
    N, DF = x.shape
    E = edge_index.shape[1]
    H = 16
    HD = e1_w1.shape[1]
    G = 64
    src = edge_index[0]
    dst = edge_index[1]

    EB = 1600
    NB = 2000
    n_eblk = E // EB
    n_nblk = N // NB

    bf = jnp.bfloat16
    w2q1 = e1_w2.reshape(HD, DF, H).transpose(1, 2, 0).reshape(DF, H * HD)
    w2q1 = w2q1.astype(bf)
    b2r1 = e1_b2.reshape(DF, H).astype(bf)
    w2q2 = e2_w2.reshape(HD, H, H).transpose(1, 2, 0).reshape(H, H * HD)
    w2q2 = w2q2.astype(bf)
    b2r2 = e2_b2.reshape(H, H).astype(bf)
    bsel = jnp.repeat(jnp.eye(H, dtype=bf), HD, axis=0)
    bns1 = (bn1_g / jnp.sqrt(1.0 + EPS)).reshape(1, H)
    bns2 = (bn2_g / jnp.sqrt(1.0 + EPS)).reshape(1, H)
    row = lambda v: v.reshape(1, -1)
    zeros128 = jnp.zeros((N, DF), jnp.float32)

    EC = E // 2
    n_eblk_c = EC // EB

    def edge_call(body, xs_c, w1, b1, w2q, b2r, din, c):
        off_blk = c * n_eblk_c
        return pl.pallas_call(
            body,
            grid=(n_eblk_c,),
            in_specs=[
                pl.BlockSpec((EB, DF), lambda i: (i, 0)),
                pl.BlockSpec((EB, 16), lambda i: (i + off_blk, 0)),
                _full((16, HD)), _full((1, HD)),
                _full((din, H * HD)), _full((din, H)), _full((H * HD, H)),
            ],
            out_specs=pl.BlockSpec((EB, DF), lambda i: (i, 0)),
            out_shape=jax.ShapeDtypeStruct((EC, DF), jnp.float32),
        )(xs_c, edge_attr, w1, b1, w2q, b2r, bsel)

    p1 = []
    for c in range(2):
        base = c * EC
        xs_c = _sc_gather(x, src, DF, base, EC)
        m1_c = edge_call(_edge1_body, xs_c, e1_w1, row(e1_b1), w2q1, b2r1,
                         DF, c)
        p1.append(_sc_scatter_add(m1_c, dst, zeros128, N, base))

    h1, inv16 = pl.pallas_call(
        _node1_body,
        grid=(n_nblk,),
        in_specs=[
            pl.BlockSpec((NC, NB, DF), lambda i: (0, i, 0)),
            pl.BlockSpec((NC, NB, DF), lambda i: (0, i, 0)),
            pl.BlockSpec((NB, DF), lambda i: (i, 0)),
            _full((DF, H)), _full((1, H)), _full((1, H)), _full((1, H)),
        ],
        out_specs=[
            pl.BlockSpec((NB, DF), lambda i: (i, 0)),
            pl.BlockSpec((NB, H), lambda i: (i, 0)),
        ],
        out_shape=[
            jax.ShapeDtypeStruct((N, DF), jnp.float32),
            jax.ShapeDtypeStruct((N, H), jnp.float32),
        ],
    )(p1[0], p1[1], x, root1, row(bias1), bns1, row(bn1_b))

    p2 = []
    for c in range(2):
        base = c * EC
        h1s_c = _sc_gather(h1, src, DF, base, EC)
        m2_c = edge_call(_edge2_body, h1s_c, e2_w1, row(e2_b1), w2q2, b2r2,
                         H, c)
        p2.append(_sc_scatter_add(m2_c, dst, zeros128, N, base))

    h2 = pl.pallas_call(
        _node2_body,
        grid=(n_nblk,),
        in_specs=[
            pl.BlockSpec((NC, NB, DF), lambda i: (0, i, 0)),
            pl.BlockSpec((NC, NB, DF), lambda i: (0, i, 0)),
            pl.BlockSpec((NB, H), lambda i: (i, 0)),
            pl.BlockSpec((NB, DF), lambda i: (i, 0)),
            _full((H, H)), _full((1, H)), _full((1, H)), _full((1, H)),
        ],
        out_specs=pl.BlockSpec((NB, H), lambda i: (i, 0)),
        out_shape=jax.ShapeDtypeStruct((N, H), jnp.float32),
    )(p2[0], p2[1], inv16, h1, root2, row(bias2), bns2, row(bn2_b))

    out = pl.pallas_call(
        _pool_body,
        in_specs=[
            pl.BlockSpec((N, H), lambda: (0, 0)),
            pl.BlockSpec((N, 1), lambda: (0, 0)),
            pl.BlockSpec((H, 8), lambda: (0, 0)),
            pl.BlockSpec((1, 8), lambda: (0, 0)),
            pl.BlockSpec((8, H), lambda: (0, 0)),
            pl.BlockSpec((1, H), lambda: (0, 0)),
        ],
        out_specs=pl.BlockSpec((G, H), lambda: (0, 0)),
        out_shape=jax.ShapeDtypeStruct((G, H), jnp.float32),
    )(h2, batch.reshape(N, 1), m_w1, row(m_b1), m_w2, row(m_b2))

    return out


# speedup vs baseline: 1.3559x; 1.0541x over previous
import functools

import jax
import jax.numpy as jnp
from jax import lax
from jax.experimental import pallas as pl
from jax.experimental.pallas import tpu as pltpu
from jax.experimental.pallas import tpu_sc as plsc

NC = 2
NS = 16
NW = NC * NS
CH = 128

EPS = 1e-5


def _sc_mesh():
    return plsc.VectorSubcoreMesh(core_axis_name="c", subcore_axis_name="s")


def _sc_gather(table, idx, D, base, ec):
    assert ec % CH == 0
    nch = ec // CH
    iters = nch // NW
    dt = table.dtype

    @functools.partial(
        pl.kernel,
        out_type=jax.ShapeDtypeStruct((ec, D), dt),
        mesh=_sc_mesh(),
        scratch_types=[
            pltpu.VMEM((CH,), jnp.int32),
            pltpu.VMEM((CH,), jnp.int32),
            pltpu.VMEM((CH, D), dt),
            pltpu.VMEM((CH, D), dt),
            pltpu.SemaphoreType.DMA,
            pltpu.SemaphoreType.DMA,
            pltpu.SemaphoreType.DMA,
            pltpu.SemaphoreType.DMA,
        ],
    )
    def k(table_hbm, idx_hbm, out_hbm, i0, i1, r0, r1, si, sg, sw0, sw1):
        wid = lax.axis_index("s") * NC + lax.axis_index("c")
        idx_v = [i0, i1]
        rows_v = [r0, r1]
        sw = [sw0, sw1]
        wpend = [None, None]

        def chunk_off(j):
            return (j * NW + wid) * CH

        pltpu.sync_copy(idx_hbm.at[pl.ds(base + chunk_off(0), CH)], idx_v[0])
        for j in range(iters):
            b = j % 2
            hi = None
            if j + 1 < iters:
                hi = pltpu.async_copy(
                    idx_hbm.at[pl.ds(base + chunk_off(j + 1), CH)],
                    idx_v[1 - b], si)
            if wpend[b] is not None:
                wpend[b].wait()
            pltpu.async_copy(table_hbm.at[idx_v[b]], rows_v[b], sg).wait()
            wpend[b] = pltpu.async_copy(
                rows_v[b], out_hbm.at[pl.ds(chunk_off(j), CH)], sw[b])
            if hi is not None:
                hi.wait()
        for b in range(2):
            if wpend[b] is not None:
                wpend[b].wait()

        if iters * NW < nch:
            @pl.when(iters * NW + wid < nch)
            def _():
                off = chunk_off(iters)
                pltpu.sync_copy(idx_hbm.at[pl.ds(base + off, CH)], idx_v[0])
                pltpu.async_copy(table_hbm.at[idx_v[0]], rows_v[0], sg).wait()
                pltpu.sync_copy(rows_v[0], out_hbm.at[pl.ds(off, CH)])

    return k(table, idx)


def _sc_scatter_add(rows, dst, zeros_nw, n, base):
    ec, W = rows.shape
    assert ec % CH == 0
    nch = ec // CH
    iters = nch // NW

    @functools.partial(
        pl.kernel,
        out_type=jax.ShapeDtypeStruct((NC, n, W), jnp.float32),
        mesh=_sc_mesh(),
        scratch_types=[
            pltpu.VMEM((CH,), jnp.int32),
            pltpu.VMEM((CH,), jnp.int32),
            pltpu.VMEM((CH, W), jnp.float32),
            pltpu.VMEM((CH, W), jnp.float32),
            pltpu.VMEM_SHARED((n, W), jnp.float32),
            pltpu.SemaphoreType.DMA,
            pltpu.SemaphoreType.DMA,
            pltpu.SemaphoreType.DMA,
            pltpu.SemaphoreType.DMA,
        ],
    )
    def k(m_hbm, dst_hbm, zer_hbm, out_hbm, i0, i1, r0, r1, acc_sh,
          si, sm, ss0, ss1):
        cid = lax.axis_index("c")
        sid = lax.axis_index("s")
        wid = sid * NC + cid
        idx_v = [i0, i1]
        rows_v = [r0, r1]
        ss = [ss0, ss1]
        spend = [None, None]

        def chunk_off(j):
            return (j * NW + wid) * CH

        rows_per_sub = n // NS
        pltpu.sync_copy(zer_hbm, rows_v[0])
        for t in range((rows_per_sub + CH - 1) // CH):
            sz = min(CH, rows_per_sub - t * CH)
            src = rows_v[0] if sz == CH else rows_v[0].at[pl.ds(0, sz)]
            pltpu.sync_copy(
                src, acc_sh.at[pl.ds(sid * rows_per_sub + t * CH, sz)])

        plsc.subcore_barrier()

        pltpu.sync_copy(dst_hbm.at[pl.ds(base + chunk_off(0), CH)], idx_v[0])
        pltpu.sync_copy(m_hbm.at[pl.ds(chunk_off(0), CH)], rows_v[0])
        for j in range(iters):
            b = j % 2
            hi = hr = None
            if j + 1 < iters:
                if spend[1 - b] is not None:
                    spend[1 - b].wait()
                    spend[1 - b] = None
                hi = pltpu.async_copy(
                    dst_hbm.at[pl.ds(base + chunk_off(j + 1), CH)],
                    idx_v[1 - b], si)
                hr = pltpu.async_copy(
                    m_hbm.at[pl.ds(chunk_off(j + 1), CH)], rows_v[1 - b], sm)
            spend[b] = pltpu.async_copy(
                rows_v[b], acc_sh.at[idx_v[b]], ss[b], add=True)
            if hi is not None:
                hi.wait()
                hr.wait()
        for b in range(2):
            if spend[b] is not None:
                spend[b].wait()

        if iters * NW < nch:
            @pl.when(iters * NW + wid < nch)
            def _():
                off = chunk_off(iters)
                pltpu.sync_copy(dst_hbm.at[pl.ds(base + off, CH)], idx_v[0])
                pltpu.sync_copy(m_hbm.at[pl.ds(off, CH)], rows_v[0])
                pltpu.sync_copy(rows_v[0], acc_sh.at[idx_v[0]], add=True)

        plsc.subcore_barrier()

        @pl.when(sid == 0)
        def _():
            pltpu.sync_copy(acc_sh, out_hbm.at[cid])

    return k(rows, dst, zeros_nw)


def _edge1_body(xs_ref, ea_ref, w1_ref, b1_ref, w2q_ref, b2r_ref, bsel_ref,
                out_ref):
    xs = xs_ref[...].astype(jnp.bfloat16)
    h = jnp.maximum(
        jnp.dot(ea_ref[...], w1_ref[...], preferred_element_type=jnp.float32)
        + b1_ref[...], 0.0)
    y = jnp.dot(xs, w2q_ref[...], preferred_element_type=jnp.float32)
    ht = jnp.concatenate([h] * 16, axis=1)
    m = (jnp.dot((ht * y).astype(jnp.bfloat16), bsel_ref[...],
                 preferred_element_type=jnp.float32)
         + jnp.dot(xs, b2r_ref[...], preferred_element_type=jnp.float32))
    eb = m.shape[0]
    cnt_cols = jnp.where(
        lax.broadcasted_iota(jnp.int32, (eb, 16), 1) == 0, 1.0, 0.0)
    out_ref[...] = jnp.concatenate(
        [m, cnt_cols, jnp.zeros((eb, 96), jnp.float32)], axis=1)


def _edge2_body(hs_ref, ea_ref, w1_ref, b1_ref, w2q_ref, b2r_ref, bsel_ref,
                out_ref):
    hs = hs_ref[:, 0:16].astype(jnp.bfloat16)
    h = jnp.maximum(
        jnp.dot(ea_ref[...], w1_ref[...], preferred_element_type=jnp.float32)
        + b1_ref[...], 0.0)
    y = jnp.dot(hs, w2q_ref[...], preferred_element_type=jnp.float32)
    ht = jnp.concatenate([h] * 16, axis=1)
    m = (jnp.dot((ht * y).astype(jnp.bfloat16), bsel_ref[...],
                 preferred_element_type=jnp.float32)
         + jnp.dot(hs, b2r_ref[...], preferred_element_type=jnp.float32))
    out_ref[...] = jnp.concatenate(
        [m, jnp.zeros((m.shape[0], 112), jnp.float32)], axis=1)


def _node1_body(pa_ref, pb_ref, x_ref, root_ref, bias_ref, bns_ref, bnb_ref,
                h_ref, inv_ref):
    p = pa_ref[0] + pa_ref[1] + pb_ref[0] + pb_ref[1]
    s = p[:, 0:16]
    cnt = p[:, 16:17]
    inv = 1.0 / jnp.maximum(cnt, 1.0)
    v = (s * inv
         + jnp.dot(x_ref[...], root_ref[...],
                   preferred_element_type=jnp.float32)
         + bias_ref[...])
    h = jnp.maximum(v * bns_ref[...] + bnb_ref[...], 0.0)
    nb = h.shape[0]
    h_ref[...] = jnp.concatenate([h, jnp.zeros((nb, 112), jnp.float32)],
                                 axis=1)
    inv_ref[...] = jnp.broadcast_to(inv, inv_ref.shape)


def _node2_body(pa_ref, pb_ref, inv_ref, h1_ref, root_ref, bias_ref, bns_ref,
                bnb_ref, out_ref):
    s = (pa_ref[0, :, 0:16] + pa_ref[1, :, 0:16]
         + pb_ref[0, :, 0:16] + pb_ref[1, :, 0:16])
    v = (s * inv_ref[...]
         + jnp.dot(h1_ref[:, 0:16], root_ref[...],
                   preferred_element_type=jnp.float32)
         + bias_ref[...])
    out_ref[...] = jnp.maximum(v * bns_ref[...] + bnb_ref[...], 0.0)


def _pool_body(h2_ref, b_ref, w1_ref, b1_ref, w2_ref, b2_ref, out_ref):
    n, _ = h2_ref.shape
    g = 64
    h2 = h2_ref[...]
    oh = jnp.where(
        b_ref[...] == lax.broadcasted_iota(jnp.int32, (n, g), 1), 1.0, 0.0)
    s = lax.dot_general(oh, h2, (((0,), (0,)), ((), ())),
                        preferred_element_type=jnp.float32)
    cnt16 = lax.dot_general(oh, jnp.ones((n, 16), jnp.float32),
                            (((0,), (0,)), ((), ())),
                            preferred_element_type=jnp.float32)
    xp = s / jnp.maximum(cnt16, 1.0)
    hm = jnp.maximum(
        jnp.dot(xp, w1_ref[...], preferred_element_type=jnp.float32)
        + b1_ref[...], 0.0)
    out_ref[...] = (
        jnp.dot(hm, w2_ref[...], preferred_element_type=jnp.float32)
        + b2_ref[...])


def _full(shape):
    return pl.BlockSpec(shape, lambda i: (0,) * len(shape))


def kernel(x, edge_index, edge_attr, batch,
           e1_w1, e1_b1, e1_w2, e1_b2, root1, bias1, bn1_g, bn1_b,
           e2_w1, e2_b1, e2_w2, e2_b2, root2, bias2, bn2_g, bn2_b,
           m_w1, m_b1, m_w2, m_b2):
    N, DF = x.shape
    E = edge_index.shape[1]
    H = 16
    HD = e1_w1.shape[1]
    G = 64
    src = edge_index[0]
    dst = edge_index[1]

    EB = 3200
    NB = 2000
    n_eblk = E // EB
    n_nblk = N // NB

    bf = jnp.bfloat16
    w2q1 = e1_w2.reshape(HD, DF, H).transpose(1, 2, 0).reshape(DF, H * HD)
    w2q1 = w2q1.astype(bf)
    b2r1 = e1_b2.reshape(DF, H).astype(bf)
    w2q2 = e2_w2.reshape(HD, H, H).transpose(1, 2, 0).reshape(H, H * HD)
    w2q2 = w2q2.astype(bf)
    b2r2 = e2_b2.reshape(H, H).astype(bf)
    bsel = jnp.repeat(jnp.eye(H, dtype=bf), HD, axis=0)
    bns1 = (bn1_g / jnp.sqrt(1.0 + EPS)).reshape(1, H)
    bns2 = (bn2_g / jnp.sqrt(1.0 + EPS)).reshape(1, H)
    row = lambda v: v.reshape(1, -1)
    zeros128 = jnp.zeros((CH, DF), jnp.float32)

    EC = E // 2
    n_eblk_c = EC // EB

    def edge_call(body, xs_c, w1, b1, w2q, b2r, din, c):
        off_blk = c * n_eblk_c
        return pl.pallas_call(
            body,
            grid=(n_eblk_c,),
            in_specs=[
                pl.BlockSpec((EB, DF), lambda i: (i, 0)),
                pl.BlockSpec((EB, 16), lambda i: (i + off_blk, 0)),
                _full((16, HD)), _full((1, HD)),
                _full((din, H * HD)), _full((din, H)), _full((H * HD, H)),
            ],
            out_specs=pl.BlockSpec((EB, DF), lambda i: (i, 0)),
            out_shape=jax.ShapeDtypeStruct((EC, DF), jnp.float32),
        )(xs_c, edge_attr, w1, b1, w2q, b2r, bsel)

    p1 = []
    for c in range(2):
        base = c * EC
        xs_c = _sc_gather(x, src, DF, base, EC)
        m1_c = edge_call(_edge1_body, xs_c, e1_w1, row(e1_b1), w2q1, b2r1,
                         DF, c)
        p1.append(_sc_scatter_add(m1_c, dst, zeros128, N, base))

    h1, inv16 = pl.pallas_call(
        _node1_body,
        grid=(n_nblk,),
        in_specs=[
            pl.BlockSpec((NC, NB, DF), lambda i: (0, i, 0)),
            pl.BlockSpec((NC, NB, DF), lambda i: (0, i, 0)),
            pl.BlockSpec((NB, DF), lambda i: (i, 0)),
            _full((DF, H)), _full((1, H)), _full((1, H)), _full((1, H)),
        ],
        out_specs=[
            pl.BlockSpec((NB, DF), lambda i: (i, 0)),
            pl.BlockSpec((NB, H), lambda i: (i, 0)),
        ],
        out_shape=[
            jax.ShapeDtypeStruct((N, DF), jnp.float32),
            jax.ShapeDtypeStruct((N, H), jnp.float32),
        ],
    )(p1[0], p1[1], x, root1, row(bias1), bns1, row(bn1_b))

    p2 = []
    for c in range(2):
        base = c * EC
        h1s_c = _sc_gather(h1, src, DF, base, EC)
        m2_c = edge_call(_edge2_body, h1s_c, e2_w1, row(e2_b1), w2q2, b2r2,
                         H, c)
        p2.append(_sc_scatter_add(m2_c, dst, zeros128, N, base))

    h2 = pl.pallas_call(
        _node2_body,
        grid=(n_nblk,),
        in_specs=[
            pl.BlockSpec((NC, NB, DF), lambda i: (0, i, 0)),
            pl.BlockSpec((NC, NB, DF), lambda i: (0, i, 0)),
            pl.BlockSpec((NB, H), lambda i: (i, 0)),
            pl.BlockSpec((NB, DF), lambda i: (i, 0)),
            _full((H, H)), _full((1, H)), _full((1, H)), _full((1, H)),
        ],
        out_specs=pl.BlockSpec((NB, H), lambda i: (i, 0)),
        out_shape=jax.ShapeDtypeStruct((N, H), jnp.float32),
    )(p2[0], p2[1], inv16, h1, root2, row(bias2), bns2, row(bn2_b))

    out = pl.pallas_call(
        _pool_body,
        in_specs=[
            pl.BlockSpec((N, H), lambda: (0, 0)),
            pl.BlockSpec((N, 1), lambda: (0, 0)),
            pl.BlockSpec((H, 8), lambda: (0, 0)),
            pl.BlockSpec((1, 8), lambda: (0, 0)),
            pl.BlockSpec((8, H), lambda: (0, 0)),
            pl.BlockSpec((1, H), lambda: (0, 0)),
        ],
        out_specs=pl.BlockSpec((G, H), lambda: (0, 0)),
        out_shape=jax.ShapeDtypeStruct((G, H), jnp.float32),
    )(h2, batch.reshape(N, 1), m_w1, row(m_b1), m_w2, row(m_b2))

    return out


# speedup vs baseline: 1.3971x; 1.0304x over previous
import functools

import jax
import jax.numpy as jnp
from jax import lax
from jax.experimental import pallas as pl
from jax.experimental.pallas import tpu as pltpu
from jax.experimental.pallas import tpu_sc as plsc

NC = 2
NS = 16
NW = NC * NS
CH = 128

EPS = 1e-5


def _sc_mesh():
    return plsc.VectorSubcoreMesh(core_axis_name="c", subcore_axis_name="s")


def _sc_gather(table, idx, D, base, ec):
    assert ec % CH == 0
    nch = ec // CH
    iters = nch // NW
    dt = table.dtype

    @functools.partial(
        pl.kernel,
        out_type=jax.ShapeDtypeStruct((ec, D), dt),
        mesh=_sc_mesh(),
        scratch_types=[
            pltpu.VMEM((CH,), jnp.int32),
            pltpu.VMEM((CH,), jnp.int32),
            pltpu.VMEM((CH, D), dt),
            pltpu.VMEM((CH, D), dt),
            pltpu.SemaphoreType.DMA,
            pltpu.SemaphoreType.DMA,
            pltpu.SemaphoreType.DMA,
            pltpu.SemaphoreType.DMA,
        ],
    )
    def k(table_hbm, idx_hbm, out_hbm, i0, i1, r0, r1, si, sg, sw0, sw1):
        wid = lax.axis_index("s") * NC + lax.axis_index("c")
        idx_v = [i0, i1]
        rows_v = [r0, r1]
        sw = [sw0, sw1]
        wpend = [None, None]

        def chunk_off(j):
            return (j * NW + wid) * CH

        pltpu.sync_copy(idx_hbm.at[pl.ds(base + chunk_off(0), CH)], idx_v[0])
        for j in range(iters):
            b = j % 2
            hi = None
            if j + 1 < iters:
                hi = pltpu.async_copy(
                    idx_hbm.at[pl.ds(base + chunk_off(j + 1), CH)],
                    idx_v[1 - b], si)
            if wpend[b] is not None:
                wpend[b].wait()
            pltpu.async_copy(table_hbm.at[idx_v[b]], rows_v[b], sg).wait()
            wpend[b] = pltpu.async_copy(
                rows_v[b], out_hbm.at[pl.ds(chunk_off(j), CH)], sw[b])
            if hi is not None:
                hi.wait()
        for b in range(2):
            if wpend[b] is not None:
                wpend[b].wait()

        if iters * NW < nch:
            @pl.when(iters * NW + wid < nch)
            def _():
                off = chunk_off(iters)
                pltpu.sync_copy(idx_hbm.at[pl.ds(base + off, CH)], idx_v[0])
                pltpu.async_copy(table_hbm.at[idx_v[0]], rows_v[0], sg).wait()
                pltpu.sync_copy(rows_v[0], out_hbm.at[pl.ds(off, CH)])

    return k(table, idx)


def _sc_scatter_add(rows, dst, zeros_nw, n, base):
    ec, W = rows.shape
    assert ec % CH == 0
    nch = ec // CH
    iters = nch // NW

    @functools.partial(
        pl.kernel,
        out_type=jax.ShapeDtypeStruct((NC, n, W), jnp.float32),
        mesh=_sc_mesh(),
        scratch_types=[
            pltpu.VMEM((CH,), jnp.int32),
            pltpu.VMEM((CH,), jnp.int32),
            pltpu.VMEM((CH, W), jnp.float32),
            pltpu.VMEM((CH, W), jnp.float32),
            pltpu.VMEM_SHARED((n, W), jnp.float32),
            pltpu.SemaphoreType.DMA,
            pltpu.SemaphoreType.DMA,
            pltpu.SemaphoreType.DMA,
            pltpu.SemaphoreType.DMA,
        ],
    )
    def k(m_hbm, dst_hbm, zer_hbm, out_hbm, i0, i1, r0, r1, acc_sh,
          si, sm, ss0, ss1):
        cid = lax.axis_index("c")
        sid = lax.axis_index("s")
        wid = sid * NC + cid
        idx_v = [i0, i1]
        rows_v = [r0, r1]
        ss = [ss0, ss1]
        spend = [None, None]

        def chunk_off(j):
            return (j * NW + wid) * CH

        rows_per_sub = n // NS
        pltpu.sync_copy(zer_hbm, rows_v[0])
        for t in range((rows_per_sub + CH - 1) // CH):
            sz = min(CH, rows_per_sub - t * CH)
            src = rows_v[0] if sz == CH else rows_v[0].at[pl.ds(0, sz)]
            pltpu.sync_copy(
                src, acc_sh.at[pl.ds(sid * rows_per_sub + t * CH, sz)])

        plsc.subcore_barrier()

        pltpu.sync_copy(dst_hbm.at[pl.ds(base + chunk_off(0), CH)], idx_v[0])
        pltpu.sync_copy(m_hbm.at[pl.ds(chunk_off(0), CH)], rows_v[0])
        for j in range(iters):
            b = j % 2
            hi = hr = None
            if j + 1 < iters:
                if spend[1 - b] is not None:
                    spend[1 - b].wait()
                    spend[1 - b] = None
                hi = pltpu.async_copy(
                    dst_hbm.at[pl.ds(base + chunk_off(j + 1), CH)],
                    idx_v[1 - b], si)
                hr = pltpu.async_copy(
                    m_hbm.at[pl.ds(chunk_off(j + 1), CH)], rows_v[1 - b], sm)
            spend[b] = pltpu.async_copy(
                rows_v[b], acc_sh.at[idx_v[b]], ss[b], add=True)
            if hi is not None:
                hi.wait()
                hr.wait()
        for b in range(2):
            if spend[b] is not None:
                spend[b].wait()

        if iters * NW < nch:
            @pl.when(iters * NW + wid < nch)
            def _():
                off = chunk_off(iters)
                pltpu.sync_copy(dst_hbm.at[pl.ds(base + off, CH)], idx_v[0])
                pltpu.sync_copy(m_hbm.at[pl.ds(off, CH)], rows_v[0])
                pltpu.sync_copy(rows_v[0], acc_sh.at[idx_v[0]], add=True)

        plsc.subcore_barrier()

        @pl.when(sid == 0)
        def _():
            pltpu.sync_copy(acc_sh, out_hbm.at[cid])

    return k(rows, dst, zeros_nw)


def _edge1_body(xs_ref, ea_ref, w1_ref, b1_ref, w2q_ref, b2r_ref, bsel_ref,
                out_ref):
    xs = xs_ref[...].astype(jnp.bfloat16)
    h = jnp.maximum(
        jnp.dot(ea_ref[...], w1_ref[...], preferred_element_type=jnp.float32)
        + b1_ref[...], 0.0)
    y = jnp.dot(xs, w2q_ref[...], preferred_element_type=jnp.float32)
    ht = jnp.concatenate([h] * 16, axis=1)
    m = (jnp.dot((ht * y).astype(jnp.bfloat16), bsel_ref[...],
                 preferred_element_type=jnp.float32)
         + jnp.dot(xs, b2r_ref[...], preferred_element_type=jnp.float32))
    eb = m.shape[0]
    cnt_cols = jnp.where(
        lax.broadcasted_iota(jnp.int32, (eb, 16), 1) == 0, 1.0, 0.0)
    out_ref[...] = jnp.concatenate(
        [m, cnt_cols, jnp.zeros((eb, 96), jnp.float32)], axis=1)


def _edge2_body(hs_ref, ea_ref, w1_ref, b1_ref, w2q_ref, b2r_ref, bsel_ref,
                out_ref):
    hs = hs_ref[:, 0:16].astype(jnp.bfloat16)
    h = jnp.maximum(
        jnp.dot(ea_ref[...], w1_ref[...], preferred_element_type=jnp.float32)
        + b1_ref[...], 0.0)
    y = jnp.dot(hs, w2q_ref[...], preferred_element_type=jnp.float32)
    ht = jnp.concatenate([h] * 16, axis=1)
    m = (jnp.dot((ht * y).astype(jnp.bfloat16), bsel_ref[...],
                 preferred_element_type=jnp.float32)
         + jnp.dot(hs, b2r_ref[...], preferred_element_type=jnp.float32))
    out_ref[...] = jnp.concatenate(
        [m, jnp.zeros((m.shape[0], 112), jnp.float32)], axis=1)


def _node1_body(pa_ref, pb_ref, pc_ref, pd_ref, x_ref, root_ref, bias_ref,
                bns_ref, bnb_ref, h_ref, inv_ref):
    p = (pa_ref[0] + pa_ref[1] + pb_ref[0] + pb_ref[1]
         + pc_ref[0] + pc_ref[1] + pd_ref[0] + pd_ref[1])
    s = p[:, 0:16]
    cnt = p[:, 16:17]
    inv = 1.0 / jnp.maximum(cnt, 1.0)
    v = (s * inv
         + jnp.dot(x_ref[...], root_ref[...],
                   preferred_element_type=jnp.float32)
         + bias_ref[...])
    h = jnp.maximum(v * bns_ref[...] + bnb_ref[...], 0.0)
    nb = h.shape[0]
    h_ref[...] = jnp.concatenate([h, jnp.zeros((nb, 112), jnp.float32)],
                                 axis=1)
    inv_ref[...] = jnp.broadcast_to(inv, inv_ref.shape)


def _node2_body(pa_ref, pb_ref, pc_ref, pd_ref, inv_ref, h1_ref, root_ref,
                bias_ref, bns_ref, bnb_ref, out_ref):
    s = (pa_ref[0, :, 0:16] + pa_ref[1, :, 0:16]
         + pb_ref[0, :, 0:16] + pb_ref[1, :, 0:16]
         + pc_ref[0, :, 0:16] + pc_ref[1, :, 0:16]
         + pd_ref[0, :, 0:16] + pd_ref[1, :, 0:16])
    v = (s * inv_ref[...]
         + jnp.dot(h1_ref[:, 0:16], root_ref[...],
                   preferred_element_type=jnp.float32)
         + bias_ref[...])
    out_ref[...] = jnp.maximum(v * bns_ref[...] + bnb_ref[...], 0.0)


def _pool_body(h2_ref, b_ref, w1_ref, b1_ref, w2_ref, b2_ref, out_ref):
    n, _ = h2_ref.shape
    g = 64
    h2 = h2_ref[...]
    oh = jnp.where(
        b_ref[...] == lax.broadcasted_iota(jnp.int32, (n, g), 1), 1.0, 0.0)
    s = lax.dot_general(oh, h2, (((0,), (0,)), ((), ())),
                        preferred_element_type=jnp.float32)
    cnt16 = lax.dot_general(oh, jnp.ones((n, 16), jnp.float32),
                            (((0,), (0,)), ((), ())),
                            preferred_element_type=jnp.float32)
    xp = s / jnp.maximum(cnt16, 1.0)
    hm = jnp.maximum(
        jnp.dot(xp, w1_ref[...], preferred_element_type=jnp.float32)
        + b1_ref[...], 0.0)
    out_ref[...] = (
        jnp.dot(hm, w2_ref[...], preferred_element_type=jnp.float32)
        + b2_ref[...])


def _full(shape):
    return pl.BlockSpec(shape, lambda i: (0,) * len(shape))


def kernel(x, edge_index, edge_attr, batch,
           e1_w1, e1_b1, e1_w2, e1_b2, root1, bias1, bn1_g, bn1_b,
           e2_w1, e2_b1, e2_w2, e2_b2, root2, bias2, bn2_g, bn2_b,
           m_w1, m_b1, m_w2, m_b2):
    N, DF = x.shape
    E = edge_index.shape[1]
    H = 16
    HD = e1_w1.shape[1]
    G = 64
    src = edge_index[0]
    dst = edge_index[1]

    EB = 3200
    NB = 2000
    n_eblk = E // EB
    n_nblk = N // NB

    bf = jnp.bfloat16
    w2q1 = e1_w2.reshape(HD, DF, H).transpose(1, 2, 0).reshape(DF, H * HD)
    w2q1 = w2q1.astype(bf)
    b2r1 = e1_b2.reshape(DF, H).astype(bf)
    w2q2 = e2_w2.reshape(HD, H, H).transpose(1, 2, 0).reshape(H, H * HD)
    w2q2 = w2q2.astype(bf)
    b2r2 = e2_b2.reshape(H, H).astype(bf)
    bsel = jnp.repeat(jnp.eye(H, dtype=bf), HD, axis=0)
    bns1 = (bn1_g / jnp.sqrt(1.0 + EPS)).reshape(1, H)
    bns2 = (bn2_g / jnp.sqrt(1.0 + EPS)).reshape(1, H)
    row = lambda v: v.reshape(1, -1)
    zeros128 = jnp.zeros((CH, DF), jnp.float32)

    CS = [38400, 38400, 38400, 44800]
    BASES = [0, 38400, 76800, 115200]

    def edge_call(body, xs_c, w1, b1, w2q, b2r, din, base, ec):
        off_blk = base // EB
        return pl.pallas_call(
            body,
            grid=(ec // EB,),
            in_specs=[
                pl.BlockSpec((EB, DF), lambda i: (i, 0)),
                pl.BlockSpec((EB, 16), lambda i: (i + off_blk, 0)),
                _full((16, HD)), _full((1, HD)),
                _full((din, H * HD)), _full((din, H)), _full((H * HD, H)),
            ],
            out_specs=pl.BlockSpec((EB, DF), lambda i: (i, 0)),
            out_shape=jax.ShapeDtypeStruct((ec, DF), jnp.float32),
        )(xs_c, edge_attr, w1, b1, w2q, b2r, bsel)

    p1 = []
    for base, ec in zip(BASES, CS):
        xs_c = _sc_gather(x, src, DF, base, ec)
        m1_c = edge_call(_edge1_body, xs_c, e1_w1, row(e1_b1), w2q1, b2r1,
                         DF, base, ec)
        p1.append(_sc_scatter_add(m1_c, dst, zeros128, N, base))

    pspec = pl.BlockSpec((NC, NB, DF), lambda i: (0, i, 0))
    h1, inv16 = pl.pallas_call(
        _node1_body,
        grid=(n_nblk,),
        in_specs=[
            pspec, pspec, pspec, pspec,
            pl.BlockSpec((NB, DF), lambda i: (i, 0)),
            _full((DF, H)), _full((1, H)), _full((1, H)), _full((1, H)),
        ],
        out_specs=[
            pl.BlockSpec((NB, DF), lambda i: (i, 0)),
            pl.BlockSpec((NB, H), lambda i: (i, 0)),
        ],
        out_shape=[
            jax.ShapeDtypeStruct((N, DF), jnp.float32),
            jax.ShapeDtypeStruct((N, H), jnp.float32),
        ],
    )(*p1, x, root1, row(bias1), bns1, row(bn1_b))

    p2 = []
    for base, ec in zip(BASES, CS):
        h1s_c = _sc_gather(h1, src, DF, base, ec)
        m2_c = edge_call(_edge2_body, h1s_c, e2_w1, row(e2_b1), w2q2, b2r2,
                         H, base, ec)
        p2.append(_sc_scatter_add(m2_c, dst, zeros128, N, base))

    h2 = pl.pallas_call(
        _node2_body,
        grid=(n_nblk,),
        in_specs=[
            pspec, pspec, pspec, pspec,
            pl.BlockSpec((NB, H), lambda i: (i, 0)),
            pl.BlockSpec((NB, DF), lambda i: (i, 0)),
            _full((H, H)), _full((1, H)), _full((1, H)), _full((1, H)),
        ],
        out_specs=pl.BlockSpec((NB, H), lambda i: (i, 0)),
        out_shape=jax.ShapeDtypeStruct((N, H), jnp.float32),
    )(*p2, inv16, h1, root2, row(bias2), bns2, row(bn2_b))

    out = pl.pallas_call(
        _pool_body,
        in_specs=[
            pl.BlockSpec((N, H), lambda: (0, 0)),
            pl.BlockSpec((N, 1), lambda: (0, 0)),
            pl.BlockSpec((H, 8), lambda: (0, 0)),
            pl.BlockSpec((1, 8), lambda: (0, 0)),
            pl.BlockSpec((8, H), lambda: (0, 0)),
            pl.BlockSpec((1, H), lambda: (0, 0)),
        ],
        out_specs=pl.BlockSpec((G, H), lambda: (0, 0)),
        out_shape=jax.ShapeDtypeStruct((G, H), jnp.float32),
    )(h2, batch.reshape(N, 1), m_w1, row(m_b1), m_w2, row(m_b2))

    return out
